# Initial kernel scaffold; baseline (speedup 1.0000x reference)
import jax
import jax.numpy as jnp
from jax.experimental import pallas as pl


def kernel(embeddings, edge_index):
    raise NotImplementedError("write your pallas kernel here")


# speedup vs baseline: 8.5139x; 8.5139x over previous
import functools

import jax
import jax.numpy as jnp
from jax import lax
from jax.experimental import pallas as pl
from jax.experimental.pallas import tpu as pltpu
from jax.experimental.pallas import tpu_sc as plsc

N_NODES = 10000
D_FEAT = 128
E_EDGES = 320000

NC, NS = 2, 16
NW = NC * NS
N_PAD = 10240
ROWS_PT = N_PAD // NS
BB = 80
NBLK_TOTAL = E_EDGES // BB
NBLK_PT = NBLK_TOTAL // NW

_sc_mesh = plsc.VectorSubcoreMesh(core_axis_name="c", subcore_axis_name="s")


@functools.partial(
    pl.kernel,
    mesh=_sc_mesh,
    out_type=(
        jax.ShapeDtypeStruct((NC, N_PAD, D_FEAT), jnp.float32),
        jax.ShapeDtypeStruct((NC, N_PAD), jnp.float32),
    ),
    scratch_types=[
        pltpu.VMEM((NBLK_PT, BB), jnp.int32),
        pltpu.VMEM((NBLK_PT, BB), jnp.int32),
        pltpu.VMEM((BB, D_FEAT), jnp.float32),
        pltpu.VMEM((BB,), jnp.float32),
        pltpu.VMEM_SHARED((N_PAD, D_FEAT), jnp.float32),
        pltpu.VMEM_SHARED((N_PAD,), jnp.float32),
        pltpu.SemaphoreType.DMA,
    ],
)
def _scatter_sum_sc(emb_hbm, src_hbm, dst_hbm, zrow_hbm, zdeg_hbm,
                    sums_out, deg_out,
                    src_v, dst_v, rows_v, ones_v, acc_sh, deg_sh, sem):
    c = lax.axis_index("c")
    s = lax.axis_index("s")
    wid = s * NC + c

    for i in range(BB // 16):
        ones_v[pl.ds(i * 16, 16)] = jnp.ones((16,), jnp.float32)

    pltpu.sync_copy(zrow_hbm, acc_sh.at[pl.ds(s * ROWS_PT, ROWS_PT)])
    pltpu.sync_copy(zdeg_hbm, deg_sh.at[pl.ds(s * ROWS_PT, ROWS_PT)])

    pltpu.sync_copy(src_hbm.at[wid], src_v)
    pltpu.sync_copy(dst_hbm.at[wid], dst_v)

    plsc.subcore_barrier()

    def blk(j, carry):
        pltpu.async_copy(emb_hbm.at[src_v.at[j]], rows_v, sem).wait()
        pltpu.sync_copy(rows_v, acc_sh.at[dst_v.at[j]], add=True)
        pltpu.sync_copy(ones_v, deg_sh.at[dst_v.at[j]], add=True)
        return carry

    lax.fori_loop(0, NBLK_PT, blk, 0)

    plsc.subcore_barrier()

    pltpu.sync_copy(acc_sh.at[pl.ds(s * ROWS_PT, ROWS_PT)],
                    sums_out.at[c, pl.ds(s * ROWS_PT, ROWS_PT)])
    pltpu.sync_copy(deg_sh.at[pl.ds(s * ROWS_PT, ROWS_PT)],
                    deg_out.at[c, pl.ds(s * ROWS_PT, ROWS_PT)])


def _combine_body(sums_ref, deg_ref, out_ref):
    s = sums_ref[0] + sums_ref[1]
    d = deg_ref[0] + deg_ref[1]
    dcol = d[:, None]
    out_ref[...] = jnp.where(dcol > 0, s / jnp.maximum(dcol, 1.0),
                             jnp.zeros_like(s))


_ROWS_BLK = 1280
_combine = pl.pallas_call(
    _combine_body,
    grid=(N_PAD // _ROWS_BLK,),
    in_specs=[
        pl.BlockSpec((NC, _ROWS_BLK, D_FEAT), lambda i: (0, i, 0)),
        pl.BlockSpec((NC, _ROWS_BLK), lambda i: (0, i)),
    ],
    out_specs=pl.BlockSpec((_ROWS_BLK, D_FEAT), lambda i: (i, 0)),
    out_shape=jax.ShapeDtypeStruct((N_PAD, D_FEAT), jnp.float32),
)


def kernel(embeddings, edge_index):
    src = edge_index[0].astype(jnp.int32).reshape(NW, NBLK_PT, BB)
    dst = edge_index[1].astype(jnp.int32).reshape(NW, NBLK_PT, BB)
    zrow = jnp.zeros((ROWS_PT, D_FEAT), jnp.float32)
    zdeg = jnp.zeros((ROWS_PT,), jnp.float32)
    sums, deg = _scatter_sum_sc(embeddings, src, dst, zrow, zdeg)
    out = _combine(sums, deg)
    return out[:N_NODES]
